# baseline (speedup 1.0000x reference)
import functools

import jax
import jax.numpy as jnp
from jax import lax
from jax.experimental import pallas as pl
from jax.experimental.pallas import tpu as pltpu
from jax.experimental.pallas import tpu_sc as plsc

N_NODES = 50000
NPAD = 51200
E_RAW = 800000
E_PAD = 819200
ROWS_PER_TILE = NPAD // 16
EDGE_ROWS = E_PAD // 128
ER_PER_TILE = EDGE_ROWS // 16
DWIN_ROWS = 8


def _sc_mesh():
    return plsc.VectorSubcoreMesh(core_axis_name="c", subcore_axis_name="s")


@functools.partial(
    pl.kernel,
    out_type=jax.ShapeDtypeStruct((2 * NPAD,), jnp.float32),
    mesh=_sc_mesh(),
    scratch_types=[
        pltpu.VMEM((DWIN_ROWS, 128), jnp.int32),
        pltpu.VMEM((DWIN_ROWS, 128), jnp.float32),
        pltpu.VMEM((800,), jnp.float32),
        pltpu.VMEM_SHARED((NPAD,), jnp.float32),
    ],
)
def _deg_kernel(dst_hbm, ew_hbm, out_hbm, d_v, w_v, zbuf, dacc):
    cid = lax.axis_index("c")
    sid = lax.axis_index("s")
    row0 = sid * ROWS_PER_TILE

    def _z(i, _):
        zbuf[pl.ds(i * 16, 16)] = jnp.zeros((16,), jnp.float32)
        return 0
    lax.fori_loop(0, 50, _z, 0, unroll=4)
    for j in range(4):
        pltpu.sync_copy(zbuf, dacc.at[pl.ds(row0 + j * 800, 800)])
    plsc.subcore_barrier()

    half_rows = EDGE_ROWS // 2
    rows_per_tile = half_rows // 16
    n_win = rows_per_tile // DWIN_ROWS

    def _win(w, _):
        base = cid * half_rows + sid * rows_per_tile + w * DWIN_ROWS
        pltpu.sync_copy(dst_hbm.at[pl.ds(base, DWIN_ROWS)], d_v)
        pltpu.sync_copy(ew_hbm.at[pl.ds(base, DWIN_ROWS)], w_v)
        for r in range(DWIN_ROWS):
            pltpu.sync_copy(w_v.at[r], dacc.at[d_v.at[r]], add=True)
        return 0
    lax.fori_loop(0, n_win, _win, 0)
    plsc.subcore_barrier()

    for j in range(4):
        pltpu.sync_copy(dacc.at[pl.ds(row0 + j * 800, 800)], zbuf)
        pltpu.sync_copy(
            zbuf, out_hbm.at[pl.ds(cid * NPAD + row0 + j * 800, 800)])


MEW = 256
MWIN_ROWS = MEW // 128
N_MWIN = ER_PER_TILE // MWIN_ROWS


def _make_msg_kernel(C):
    slots = C + (C % 2)

    @functools.partial(
        pl.kernel,
        out_type=jax.ShapeDtypeStruct((NPAD, slots, 32), jnp.float32),
        mesh=_sc_mesh(),
        scratch_types=[
            pltpu.VMEM((2, MWIN_ROWS, 128), jnp.int32),
            pltpu.VMEM((3, MWIN_ROWS, 128), jnp.int32),
            pltpu.VMEM((2, MWIN_ROWS, 128), jnp.float32),
            pltpu.VMEM((2, MEW, 32), jnp.float32),
            pltpu.VMEM((160, 32), jnp.float32),
            pltpu.VMEM_SHARED((NPAD, 32), jnp.float32),
            pltpu.SemaphoreType.DMA,
            pltpu.SemaphoreType.DMA,
            pltpu.SemaphoreType.DMA,
        ],
        compiler_params=pltpu.CompilerParams(use_tc_tiling_on_sc=False),
    )
    def _msg(h2s_hbm, src_hbm, dst_hbm, ew_hbm, out_hbm,
             s_v, d_v, w_v, rows_v, z_v, acc, sem_w, sem_g, sem_s):
        cid = lax.axis_index("c")
        sid = lax.axis_index("s")
        row0 = sid * ROWS_PER_TILE

        def _load_windows(w, p2, p3):
            base = sid * ER_PER_TILE + w * MWIN_ROWS
            pltpu.async_copy(src_hbm.at[pl.ds(base, MWIN_ROWS)],
                             s_v.at[p2], sem_w)
            pltpu.async_copy(dst_hbm.at[pl.ds(base, MWIN_ROWS)],
                             d_v.at[p3], sem_w)
            pltpu.async_copy(ew_hbm.at[pl.ds(base, MWIN_ROWS)],
                             w_v.at[p2], sem_w)

        def _drain_windows():
            for _ in range(3):
                pltpu.make_async_copy(
                    src_hbm.at[pl.ds(0, MWIN_ROWS)], s_v.at[0], sem_w).wait()

        def _sidx(p2, cc):
            for r in range(MWIN_ROWS):
                for k in range(8):
                    s_v[p2, r, pl.ds(k * 16, 16)] = (
                        s_v[p2, r, pl.ds(k * 16, 16)] * C + cc)

        def _issue_gather(p2):
            for r in range(MWIN_ROWS):
                pltpu.async_copy(
                    h2s_hbm.at[s_v.at[p2].at[r]],
                    rows_v.at[p2].at[pl.ds(r * 128, 128)], sem_g)

        def _drain_gather(p2):
            for r in range(MWIN_ROWS):
                pltpu.make_async_copy(
                    h2s_hbm.at[s_v.at[p2].at[r]],
                    rows_v.at[p2].at[pl.ds(r * 128, 128)], sem_g).wait()

        def _issue_scatter(p2, p3):
            for r in range(MWIN_ROWS):
                pltpu.async_copy(rows_v.at[p2].at[pl.ds(r * 128, 128)],
                                 acc.at[d_v.at[p3].at[r]], sem_s, add=True)

        def _drain_scatter(p2, p3):
            for r in range(MWIN_ROWS):
                pltpu.make_async_copy(
                    rows_v.at[p2].at[pl.ds(r * 128, 128)],
                    acc.at[d_v.at[p3].at[r]], sem_s).wait()

        def _run_chunk(cc, slot, w0, nw):
            def _z(i, _):
                z_v[pl.ds(i * 16, 16), :] = jnp.zeros((16, 32), jnp.float32)
                return 0
            lax.fori_loop(0, 10, _z, 0, unroll=4)
            for j in range(20):
                pltpu.sync_copy(z_v, acc.at[pl.ds(row0 + j * 160, 160)])
            plsc.subcore_barrier()

            _load_windows(w0, 0, 0)
            _drain_windows()
            _sidx(0, cc)
            _issue_gather(0)
            _load_windows(w0 + 1, 1, 1)

            def _win(w, _):
                p2 = lax.rem(w, 2)
                p3 = lax.rem(w, 3)
                _drain_gather(p2)

                def _mul(blk, _):
                    r = blk // 8
                    k16 = blk % 8
                    w16 = w_v[p2, r, pl.ds(k16 * 16, 16)]
                    e0 = blk * 16
                    for j in range(16):
                        wt = w16[j]
                        rows_v[p2, e0 + j, pl.ds(0, 16)] = (
                            rows_v[p2, e0 + j, pl.ds(0, 16)] * wt)
                        rows_v[p2, e0 + j, pl.ds(16, 16)] = (
                            rows_v[p2, e0 + j, pl.ds(16, 16)] * wt)
                    return 0
                lax.fori_loop(0, MEW // 16, _mul, 0)

                _issue_scatter(p2, p3)

                @pl.when(w + 1 < nw)
                def _next():
                    @pl.when(w >= 1)
                    def _ds():
                        _drain_scatter(1 - p2, lax.rem(w - 1, 3))
                    _drain_windows()
                    _sidx(1 - p2, cc)
                    _issue_gather(1 - p2)

                    @pl.when(w + 2 < nw)
                    def _next2():
                        _load_windows(w0 + w + 2, p2, lax.rem(w + 2, 3))
                return 0
            lax.fori_loop(0, nw, _win, 0)

            _drain_scatter(lax.rem(nw - 2, 2), lax.rem(nw - 2, 3))
            _drain_scatter(lax.rem(nw - 1, 2), lax.rem(nw - 1, 3))
            plsc.subcore_barrier()

            for j in range(20):
                pltpu.sync_copy(acc.at[pl.ds(row0 + j * 160, 160)], z_v)
                pltpu.sync_copy(
                    z_v, out_hbm.at[pl.ds(row0 + j * 160, 160), slot])

        def _task(t, _):
            cc = 2 * t + cid
            _run_chunk(cc, cc, 0, N_MWIN)
            return 0
        lax.fori_loop(0, (C - (C % 2)) // 2, _task, 0)
        if C % 2:
            _run_chunk(C - 1, C - 1 + cid, cid * (N_MWIN // 2), N_MWIN // 2)

    return _msg


_MSG_KERNELS = {C: _make_msg_kernel(C) for C in (3, 6, 11)}


def _pad2(a, rows, cols):
    return jnp.pad(a, ((0, rows - a.shape[0]), (0, cols - a.shape[1])))


def kernel(x, edge_index, edge_attr, batch, target, params):
    B = target.shape[0]

    src = edge_index[0]
    dst = edge_index[1]
    pad_e = E_PAD - E_RAW
    spread = jnp.arange(pad_e, dtype=jnp.int32) % jnp.int32(1024)
    src_p = jnp.concatenate([src, spread]).reshape(EDGE_ROWS, 128)
    dst_p = jnp.concatenate([dst, spread]).reshape(EDGE_ROWS, 128)
    ew_p = jnp.concatenate(
        [edge_attr, jnp.zeros((pad_e,), jnp.float32)]).reshape(EDGE_ROWS, 128)

    degp = _deg_kernel(dst_p, ew_p).reshape(2, NPAD)
    deg = degp[0][:N_NODES] + degp[1][:N_NODES] + 1.0
    dis = lax.rsqrt(deg)

    def gcn_layer(h, Wp, bp, C):
        h2s = (h @ Wp) * dis[:, None]
        out = _MSG_KERNELS[C](h2s.reshape(N_NODES * C, 32),
                              src_p, dst_p, ew_p)
        out2 = out.reshape(NPAD, -1)[:N_NODES]
        Af = out2[:, :C * 32]
        if C % 2:
            Af = Af + jnp.pad(out2[:, C * 32:], ((0, 0), ((C - 1) * 32, 0)))
        return jax.nn.relu((Af + h2s) * dis[:, None] + bp)

    W1, b1 = params['gcn1']
    W2, b2 = params['gcn2']
    W3, b3 = params['gcn3']
    g = gcn_layer(x, jnp.pad(W1, ((0, 0), (0, 9))), jnp.pad(b1, (0, 9)), 3)
    g = gcn_layer(g, _pad2(W2, 96, 192), jnp.pad(b2, (0, 18)), 6)
    g = gcn_layer(g, _pad2(W3, 192, 352), jnp.pad(b3, (0, 4)), 11)
    g = g[:, :348]

    g = jax.ops.segment_max(g, batch, num_segments=B)

    emb = params['embed'][target]
    h = jnp.transpose(emb, (0, 2, 1))
    feats = []
    for bi in range(3):
        t = h
        for li in range(bi + 1):
            Wc, bc = params['conv_%d_%d' % (bi, li)]
            t = lax.conv_general_dilated(
                t, Wc, (1,), 'VALID',
                dimension_numbers=('NCH', 'OIH', 'NCH')) + bc[None, :, None]
            t = jax.nn.relu(t)
        feats.append(jnp.max(t, axis=-1))
    pcat = jnp.concatenate(feats, axis=-1)
    Wl, bl = params['prot_linear']
    protein_x = pcat @ Wl.T + bl

    Wg1, bg1 = params['fc_g1']
    Wg2, bg2 = params['fc_g2']
    g = jax.nn.relu(g @ Wg1.T + bg1)
    ligand_x = g @ Wg2.T + bg2
    Wv, bv = params['v_net']
    Wq, bq = params['q_net']
    v_ = jax.nn.relu(ligand_x @ Wv.T + bv)
    q_ = jax.nn.relu(protein_x @ Wq.T + bq)
    hm = params['h_mat'].reshape(1, 576)
    att = lax.dot_general(v_ * hm, q_, (((1,), (1,)), ((), ())))
    att = (att + params['h_bias'].reshape(1, 1))[None, None]
    cx = jnp.concatenate([protein_x, ligand_x], axis=-1)
    for nm in ('cls1', 'cls2', 'cls3'):
        W, b = params[nm]
        cx = jax.nn.relu(cx @ W.T + b)
    W, b = params['cls4']
    score = cx @ W.T + b
    return (ligand_x, protein_x, att, score)


# speedup vs baseline: 1.1856x; 1.1856x over previous
import functools

import jax
import jax.numpy as jnp
from jax import lax
from jax.experimental import pallas as pl
from jax.experimental.pallas import tpu as pltpu
from jax.experimental.pallas import tpu_sc as plsc

N_NODES = 50000
NPAD = 51200
E_RAW = 800000
E_PAD = 819200
ROWS_PER_TILE = NPAD // 16
EDGE_ROWS = E_PAD // 128
ER_PER_TILE = EDGE_ROWS // 16
DWIN_ROWS = 8


def _sc_mesh():
    return plsc.VectorSubcoreMesh(core_axis_name="c", subcore_axis_name="s")


@functools.partial(
    pl.kernel,
    out_type=jax.ShapeDtypeStruct((2 * NPAD,), jnp.float32),
    mesh=_sc_mesh(),
    scratch_types=[
        pltpu.VMEM((DWIN_ROWS, 128), jnp.int32),
        pltpu.VMEM((DWIN_ROWS, 128), jnp.float32),
        pltpu.VMEM((800,), jnp.float32),
        pltpu.VMEM_SHARED((NPAD,), jnp.float32),
    ],
)
def _deg_kernel(dst_hbm, ew_hbm, out_hbm, d_v, w_v, zbuf, dacc):
    cid = lax.axis_index("c")
    sid = lax.axis_index("s")
    row0 = sid * ROWS_PER_TILE

    def _z(i, _):
        zbuf[pl.ds(i * 16, 16)] = jnp.zeros((16,), jnp.float32)
        return 0
    lax.fori_loop(0, 50, _z, 0, unroll=4)
    for j in range(4):
        pltpu.sync_copy(zbuf, dacc.at[pl.ds(row0 + j * 800, 800)])
    plsc.subcore_barrier()

    half_rows = EDGE_ROWS // 2
    rows_per_tile = half_rows // 16
    n_win = rows_per_tile // DWIN_ROWS

    def _win(w, _):
        base = cid * half_rows + sid * rows_per_tile + w * DWIN_ROWS
        pltpu.sync_copy(dst_hbm.at[pl.ds(base, DWIN_ROWS)], d_v)
        pltpu.sync_copy(ew_hbm.at[pl.ds(base, DWIN_ROWS)], w_v)
        for r in range(DWIN_ROWS):
            pltpu.sync_copy(w_v.at[r], dacc.at[d_v.at[r]], add=True)
        return 0
    lax.fori_loop(0, n_win, _win, 0)
    plsc.subcore_barrier()

    for j in range(4):
        pltpu.sync_copy(dacc.at[pl.ds(row0 + j * 800, 800)], zbuf)
        pltpu.sync_copy(
            zbuf, out_hbm.at[pl.ds(cid * NPAD + row0 + j * 800, 800)])


MEW = 256
MWIN_ROWS = MEW // 128
N_MWIN = ER_PER_TILE // MWIN_ROWS


def _make_msg_kernel(C):
    slots = C + (C % 2)

    @functools.partial(
        pl.kernel,
        out_type=jax.ShapeDtypeStruct((NPAD, slots, 32), jnp.float32),
        mesh=_sc_mesh(),
        scratch_types=[
            pltpu.VMEM((2, MWIN_ROWS, 128), jnp.int32),
            pltpu.VMEM((3, MWIN_ROWS, 128), jnp.int32),
            pltpu.VMEM((2, MWIN_ROWS, 128), jnp.float32),
            pltpu.VMEM((2, MEW, 32), jnp.float32),
            pltpu.VMEM((160, 32), jnp.float32),
            pltpu.VMEM_SHARED((NPAD, 32), jnp.float32),
            pltpu.SemaphoreType.DMA,
            pltpu.SemaphoreType.DMA,
            pltpu.SemaphoreType.DMA,
        ],
        compiler_params=pltpu.CompilerParams(use_tc_tiling_on_sc=False),
    )
    def _msg(h2s_hbm, src_hbm, dst_hbm, ew_hbm, out_hbm,
             s_v, d_v, w_v, rows_v, z_v, acc, sem_w, sem_g, sem_s):
        cid = lax.axis_index("c")
        sid = lax.axis_index("s")
        row0 = sid * ROWS_PER_TILE

        def _load_windows(w, p2, p3):
            base = sid * ER_PER_TILE + w * MWIN_ROWS
            pltpu.async_copy(src_hbm.at[pl.ds(base, MWIN_ROWS)],
                             s_v.at[p2], sem_w)
            pltpu.async_copy(dst_hbm.at[pl.ds(base, MWIN_ROWS)],
                             d_v.at[p3], sem_w)
            pltpu.async_copy(ew_hbm.at[pl.ds(base, MWIN_ROWS)],
                             w_v.at[p2], sem_w)

        def _drain_windows():
            for _ in range(3):
                pltpu.make_async_copy(
                    src_hbm.at[pl.ds(0, MWIN_ROWS)], s_v.at[0], sem_w).wait()

        def _sidx(p2, cc):
            for r in range(MWIN_ROWS):
                for k in range(8):
                    s_v[p2, r, pl.ds(k * 16, 16)] = (
                        s_v[p2, r, pl.ds(k * 16, 16)] * C + cc)

        def _issue_gather(p2):
            for r in range(MWIN_ROWS):
                pltpu.async_copy(
                    h2s_hbm.at[s_v.at[p2].at[r]],
                    rows_v.at[p2].at[pl.ds(r * 128, 128)], sem_g)

        def _drain_gather(p2):
            for r in range(MWIN_ROWS):
                pltpu.make_async_copy(
                    h2s_hbm.at[s_v.at[p2].at[r]],
                    rows_v.at[p2].at[pl.ds(r * 128, 128)], sem_g).wait()

        def _issue_scatter(p2, p3):
            for r in range(MWIN_ROWS):
                pltpu.async_copy(rows_v.at[p2].at[pl.ds(r * 128, 128)],
                                 acc.at[d_v.at[p3].at[r]], sem_s, add=True)

        def _drain_scatter(p2, p3):
            for r in range(MWIN_ROWS):
                pltpu.make_async_copy(
                    rows_v.at[p2].at[pl.ds(r * 128, 128)],
                    acc.at[d_v.at[p3].at[r]], sem_s).wait()

        def _run_chunk(cc, slot, w0, nw):
            def _z(i, _):
                z_v[pl.ds(i * 16, 16), :] = jnp.zeros((16, 32), jnp.float32)
                return 0
            lax.fori_loop(0, 10, _z, 0, unroll=4)
            for j in range(20):
                pltpu.sync_copy(z_v, acc.at[pl.ds(row0 + j * 160, 160)])
            plsc.subcore_barrier()

            _load_windows(w0, 0, 0)
            _drain_windows()
            _sidx(0, cc)
            _issue_gather(0)
            _load_windows(w0 + 1, 1, 1)

            def _win(w, _):
                p2 = lax.rem(w, 2)
                p3 = lax.rem(w, 3)
                _drain_gather(p2)

                def _mul(blk, _):
                    r = blk // 8
                    k16 = blk % 8
                    w16 = w_v[p2, r, pl.ds(k16 * 16, 16)]
                    e0 = blk * 16
                    for j in range(16):
                        wt = w16[j]
                        rows_v[p2, e0 + j, pl.ds(0, 16)] = (
                            rows_v[p2, e0 + j, pl.ds(0, 16)] * wt)
                        rows_v[p2, e0 + j, pl.ds(16, 16)] = (
                            rows_v[p2, e0 + j, pl.ds(16, 16)] * wt)
                    return 0
                lax.fori_loop(0, MEW // 16, _mul, 0)

                _issue_scatter(p2, p3)

                @pl.when(w + 1 < nw)
                def _next():
                    @pl.when(w >= 1)
                    def _ds():
                        _drain_scatter(1 - p2, lax.rem(w - 1, 3))
                    _drain_windows()
                    _sidx(1 - p2, cc)
                    _issue_gather(1 - p2)

                    @pl.when(w + 2 < nw)
                    def _next2():
                        _load_windows(w0 + w + 2, p2, lax.rem(w + 2, 3))
                return 0
            lax.fori_loop(0, nw, _win, 0)

            _drain_scatter(lax.rem(nw - 2, 2), lax.rem(nw - 2, 3))
            _drain_scatter(lax.rem(nw - 1, 2), lax.rem(nw - 1, 3))
            plsc.subcore_barrier()

            for j in range(20):
                pltpu.sync_copy(acc.at[pl.ds(row0 + j * 160, 160)], z_v)
                pltpu.sync_copy(
                    z_v, out_hbm.at[pl.ds(row0 + j * 160, 160), slot])

        def _task(t, _):
            cc = 2 * t + cid
            _run_chunk(cc, cc, 0, N_MWIN)
            return 0
        lax.fori_loop(0, (C - (C % 2)) // 2, _task, 0)
        if C % 2:
            _run_chunk(C - 1, C - 1 + cid, cid * (N_MWIN // 2), N_MWIN // 2)

    return _msg


_MSG_KERNELS = {C: _make_msg_kernel(C) for C in (3, 6, 11)}


def _pad2(a, rows, cols):
    return jnp.pad(a, ((0, rows - a.shape[0]), (0, cols - a.shape[1])))


def kernel(x, edge_index, edge_attr, batch, target, params):
    B = target.shape[0]

    src = edge_index[0]
    dst = edge_index[1]
    pad_e = E_PAD - E_RAW
    spread = jnp.arange(pad_e, dtype=jnp.int32) % jnp.int32(1024)
    src_p = jnp.concatenate([src, spread]).reshape(EDGE_ROWS, 128)
    dst_p = jnp.concatenate([dst, spread]).reshape(EDGE_ROWS, 128)
    ew_p = jnp.concatenate(
        [edge_attr, jnp.zeros((pad_e,), jnp.float32)]).reshape(EDGE_ROWS, 128)

    xp = jnp.pad(x, ((0, NPAD - N_NODES), (0, 96 - 87)))

    degp = _deg_kernel(dst_p, ew_p).reshape(2, NPAD)
    deg = degp[0] + degp[1] + 1.0
    dis = lax.rsqrt(deg)

    def gcn_layer(h, Wp, bp, C):
        h2s = (h @ Wp) * dis[:, None]
        out = _MSG_KERNELS[C](h2s.reshape(NPAD * C, 32),
                              src_p, dst_p, ew_p)
        out2 = out.reshape(NPAD, -1)
        Af = out2[:, :C * 32]
        if C % 2:
            Af = Af + jnp.pad(out2[:, C * 32:], ((0, 0), ((C - 1) * 32, 0)))
        return jax.nn.relu((Af + h2s) * dis[:, None] + bp)

    W1, b1 = params['gcn1']
    W2, b2 = params['gcn2']
    W3, b3 = params['gcn3']
    g = gcn_layer(xp, _pad2(W1, 96, 96), jnp.pad(b1, (0, 9)), 3)
    g = gcn_layer(g, _pad2(W2, 96, 192), jnp.pad(b2, (0, 18)), 6)
    g = gcn_layer(g, _pad2(W3, 192, 352), jnp.pad(b3, (0, 4)), 11)
    g = g[:N_NODES, :348]

    g = jax.ops.segment_max(g, batch, num_segments=B)

    emb = params['embed'][target]
    h = jnp.transpose(emb, (0, 2, 1))
    feats = []
    for bi in range(3):
        t = h
        for li in range(bi + 1):
            Wc, bc = params['conv_%d_%d' % (bi, li)]
            t = lax.conv_general_dilated(
                t, Wc, (1,), 'VALID',
                dimension_numbers=('NCH', 'OIH', 'NCH')) + bc[None, :, None]
            t = jax.nn.relu(t)
        feats.append(jnp.max(t, axis=-1))
    pcat = jnp.concatenate(feats, axis=-1)
    Wl, bl = params['prot_linear']
    protein_x = pcat @ Wl.T + bl

    Wg1, bg1 = params['fc_g1']
    Wg2, bg2 = params['fc_g2']
    g = jax.nn.relu(g @ Wg1.T + bg1)
    ligand_x = g @ Wg2.T + bg2
    Wv, bv = params['v_net']
    Wq, bq = params['q_net']
    v_ = jax.nn.relu(ligand_x @ Wv.T + bv)
    q_ = jax.nn.relu(protein_x @ Wq.T + bq)
    hm = params['h_mat'].reshape(1, 576)
    att = lax.dot_general(v_ * hm, q_, (((1,), (1,)), ((), ())))
    att = (att + params['h_bias'].reshape(1, 1))[None, None]
    cx = jnp.concatenate([protein_x, ligand_x], axis=-1)
    for nm in ('cls1', 'cls2', 'cls3'):
        W, b = params[nm]
        cx = jax.nn.relu(cx @ W.T + b)
    W, b = params['cls4']
    score = cx @ W.T + b
    return (ligand_x, protein_x, att, score)
